# baseline (speedup 1.0000x reference)
import functools

import jax
import jax.numpy as jnp
from jax import lax
from jax.experimental import pallas as pl
from jax.experimental.pallas import tpu as pltpu

_K_CHUNKS = 1


def _masked_bce_tc(p_view, ind, target_t, maskf, win_sq):
    B, W, S = p_view.shape
    N = ind.shape[-1]
    chunk = S // _K_CHUNKS

    def body(p_ref, i_ref, t_ref, m_ref, o_ref, acc_ref):
        k = pl.program_id(1)

        @pl.when(k == 0)
        def _():
            acc_ref[...] = jnp.zeros_like(acc_ref)

        indv = i_ref[0, 0]
        local = indv - k * chunk
        oh = (lax.broadcasted_iota(jnp.int32, (N, chunk), 1)
              == local[:, None]).astype(jnp.float32)
        p = p_ref[0]
        acc_ref[...] += lax.dot_general(
            p, oh, (((1,), (1,)), ((), ())),
            preferred_element_type=jnp.float32)

        @pl.when(k == _K_CHUNKS - 1)
        def _():
            pred = acc_ref[...]
            t = t_ref[0]
            m = m_ref[0]
            c = jnp.log1p(jnp.exp(-jnp.abs(pred)))
            mn = jnp.minimum(pred, 0.0)
            ls_p = mn - c
            ls_mp = mn - pred - c
            bce = -(1.5 * t * ls_p + (1.0 - t) * ls_mp)
            o_ref[0, 0, 0] = jnp.sum(bce * m)
            o_ref[0, 0, 1] = jnp.sum(m) * float(win_sq)

    grid = (B, _K_CHUNKS)
    return pl.pallas_call(
        body,
        grid=grid,
        in_specs=[
            pl.BlockSpec((1, W, chunk), lambda b, k: (b, 0, k)),
            pl.BlockSpec((1, 1, N), lambda b, k: (b, 0, 0)),
            pl.BlockSpec((1, W, N), lambda b, k: (b, 0, 0)),
            pl.BlockSpec((1, 1, N), lambda b, k: (b, 0, 0)),
        ],
        out_specs=pl.BlockSpec((1, 1, 2), lambda b, k: (b, 0, 0),
                               memory_space=pltpu.SMEM),
        out_shape=jax.ShapeDtypeStruct((B, 1, 2), jnp.float32),
        scratch_shapes=[pltpu.VMEM((W, N), jnp.float32)],
    )(p_view, ind, target_t, maskf)


def kernel(output, mask, ind, target):
    B, S, W = output.shape
    N = ind.shape[1]
    win_sq = target.shape[-1] * target.shape[-2]
    p_view = output.transpose(0, 2, 1)
    target_t = target.reshape(B, N, win_sq).transpose(0, 2, 1)
    maskf = mask.astype(jnp.float32).reshape(B, 1, N)
    parts = _masked_bce_tc(p_view, ind.reshape(B, 1, N), target_t, maskf, win_sq)
    loss_sum = jnp.sum(parts[:, 0, 0])
    num_sample = jnp.sum(parts[:, 0, 1])
    return jnp.where(num_sample > 0, loss_sum / num_sample, loss_sum)
